# Initial kernel scaffold; baseline (speedup 1.0000x reference)
import jax
import jax.numpy as jnp
from jax.experimental import pallas as pl


def kernel(token_type_ids, sem_ids, seq_mask, emb_weight):
    raise NotImplementedError("write your pallas kernel here")


# speedup vs baseline: 1.8793x; 1.8793x over previous
import functools

import jax
import jax.numpy as jnp
from jax import lax
from jax.experimental import pallas as pl
from jax.experimental.pallas import tpu as pltpu
from jax.experimental.pallas import tpu_sc as plsc

NUM_EMBEDDINGS = 100000
EMBED_DIM = 64
PADDING_IDX = NUM_EMBEDDINGS * 4

_LANES = 16
_NW = 32
_CHUNK = 512
_GATHER = 128
_NGATHER = _CHUNK // _GATHER


def _make_lookup(n):
    assert n % (_NW * _CHUNK) == 0
    per_w = n // _NW
    steps = per_w // _CHUNK
    mesh = plsc.VectorSubcoreMesh(core_axis_name="c", subcore_axis_name="s")

    @functools.partial(
        pl.kernel,
        mesh=mesh,
        out_type=jax.ShapeDtypeStruct((n, EMBED_DIM), jnp.float32),
        scratch_types=[
            pltpu.VMEM((_CHUNK,), jnp.int32),
            pltpu.VMEM((_CHUNK,), jnp.int32),
            pltpu.VMEM((_CHUNK,), jnp.int32),
            pltpu.VMEM((_NGATHER, _GATHER), jnp.int32),
            pltpu.VMEM((_CHUNK, EMBED_DIM), jnp.float32),
            pltpu.SemaphoreType.DMA,
        ],
        compiler_params=pltpu.CompilerParams(use_tc_tiling_on_sc=False),
    )
    def lookup(tt_h, sem_h, mk_h, tab_h, out_h,
               tt_v, sem_v, mk_v, idx_v, rows_v, dsem):
        wid = lax.axis_index("s") * 2 + lax.axis_index("c")
        wbase = wid * per_w

        def step(g, carry):
            base = wbase + g * _CHUNK
            pltpu.sync_copy(tt_h.at[pl.ds(base, _CHUNK)], tt_v)
            pltpu.sync_copy(sem_h.at[pl.ds(base, _CHUNK)], sem_v)
            pltpu.sync_copy(mk_h.at[pl.ds(base, _CHUNK)], mk_v)
            for i in range(_CHUNK // _LANES):
                sl = pl.ds(i * _LANES, _LANES)
                ids = tt_v[sl] * NUM_EMBEDDINGS + sem_v[sl]
                ids = jnp.where(mk_v[sl] != 0, ids, PADDING_IDX)
                idx_v[i // (_GATHER // _LANES),
                      pl.ds((i % (_GATHER // _LANES)) * _LANES, _LANES)] = ids
            copies = [
                pltpu.async_copy(
                    tab_h.at[idx_v.at[j]],
                    rows_v.at[pl.ds(j * _GATHER, _GATHER)],
                    dsem,
                )
                for j in range(_NGATHER)
            ]
            for c in copies:
                c.wait()
            pltpu.sync_copy(rows_v, out_h.at[pl.ds(base, _CHUNK)])
            return carry

        lax.fori_loop(0, steps, step, 0)

    return lookup


def kernel(token_type_ids, sem_ids, seq_mask, emb_weight):
    b, l = token_type_ids.shape
    n = b * l
    tt = token_type_ids.reshape(n).astype(jnp.int32)
    sem = sem_ids.reshape(n).astype(jnp.int32)
    mk = seq_mask.reshape(n).astype(jnp.int32)
    out = _make_lookup(n)(tt, sem, mk, emb_weight)
    return out.reshape(b, l, EMBED_DIM)


# speedup vs baseline: 1.8796x; 1.0002x over previous
import functools

import jax
import jax.numpy as jnp
from jax import lax
from jax.experimental import pallas as pl
from jax.experimental.pallas import tpu as pltpu
from jax.experimental.pallas import tpu_sc as plsc

NUM_EMBEDDINGS = 100000
EMBED_DIM = 64
PADDING_IDX = NUM_EMBEDDINGS * 4

_LANES = 16
_NW = 32
_CHUNK = 512
_GATHER = 128
_NGATHER = _CHUNK // _GATHER
_NBUF = 2


def _make_lookup(n):
    assert n % (_NW * _CHUNK * _NBUF) == 0
    per_w = n // _NW
    steps = per_w // _CHUNK
    pairs = steps // _NBUF
    mesh = plsc.VectorSubcoreMesh(core_axis_name="c", subcore_axis_name="s")

    @functools.partial(
        pl.kernel,
        mesh=mesh,
        out_type=jax.ShapeDtypeStruct((n, EMBED_DIM), jnp.float32),
        scratch_types=[
            pltpu.VMEM((_NBUF, _CHUNK), jnp.int32),
            pltpu.VMEM((_NBUF, _CHUNK), jnp.int32),
            pltpu.VMEM((_NBUF, _CHUNK), jnp.int32),
            pltpu.VMEM((_NBUF, _NGATHER, _GATHER), jnp.int32),
            pltpu.VMEM((_NBUF, _CHUNK, EMBED_DIM), jnp.float32),
            pltpu.SemaphoreType.DMA,
            pltpu.SemaphoreType.DMA,
            pltpu.SemaphoreType.DMA,
            pltpu.SemaphoreType.DMA,
            pltpu.SemaphoreType.DMA,
            pltpu.SemaphoreType.DMA,
        ],
        compiler_params=pltpu.CompilerParams(use_tc_tiling_on_sc=False),
    )
    def lookup(tt_h, sem_h, mk_h, tab_h, out_h,
               tt_v, sem_v, mk_v, idx_v, rows_v,
               in_s0, in_s1, g_s0, g_s1, w_s0, w_s1):
        in_s = [in_s0, in_s1]
        g_s = [g_s0, g_s1]
        w_s = [w_s0, w_s1]
        wid = lax.axis_index("s") * 2 + lax.axis_index("c")
        wbase = wid * per_w

        def fire_inputs(g, b):
            base = wbase + g * _CHUNK
            pltpu.async_copy(tt_h.at[pl.ds(base, _CHUNK)], tt_v.at[b], in_s[b])
            pltpu.async_copy(sem_h.at[pl.ds(base, _CHUNK)], sem_v.at[b], in_s[b])
            pltpu.async_copy(mk_h.at[pl.ds(base, _CHUNK)], mk_v.at[b], in_s[b])

        def drain_inputs(b):
            sl = pl.ds(wbase, _CHUNK)
            pltpu.make_async_copy(tt_h.at[sl], tt_v.at[b], in_s[b]).wait()
            pltpu.make_async_copy(sem_h.at[sl], sem_v.at[b], in_s[b]).wait()
            pltpu.make_async_copy(mk_h.at[sl], mk_v.at[b], in_s[b]).wait()

        def compute_idx(b):
            for i in range(_CHUNK // _LANES):
                sl = pl.ds(i * _LANES, _LANES)
                ids = tt_v[b, sl] * NUM_EMBEDDINGS + sem_v[b, sl]
                ids = jnp.where(mk_v[b, sl] != 0, ids, PADDING_IDX)
                idx_v[b, i // (_GATHER // _LANES),
                      pl.ds((i % (_GATHER // _LANES)) * _LANES, _LANES)] = ids

        def fire_gathers(b):
            for j in range(_NGATHER):
                pltpu.async_copy(
                    tab_h.at[idx_v.at[b, j]],
                    rows_v.at[b, pl.ds(j * _GATHER, _GATHER)],
                    g_s[b],
                )

        def drain_gathers(b):
            for j in range(_NGATHER):
                pltpu.make_async_copy(
                    tab_h.at[idx_v.at[b, j]],
                    rows_v.at[b, pl.ds(j * _GATHER, _GATHER)],
                    g_s[b],
                ).wait()

        def fire_wb(g, b):
            base = wbase + g * _CHUNK
            pltpu.async_copy(rows_v.at[b], out_h.at[pl.ds(base, _CHUNK)], w_s[b])

        def drain_wb(b):
            pltpu.make_async_copy(
                rows_v.at[b], out_h.at[pl.ds(wbase, _CHUNK)], w_s[b]).wait()

        def steady(g, b):
            b1 = 1 - b
            drain_inputs(b)
            compute_idx(b)
            gnext = jnp.where(g + 1 < steps, g + 1, 0)
            fire_inputs(gnext, b1)
            drain_gathers(b1)
            fire_wb(g - 1, b1)
            drain_wb(b)
            fire_gathers(b)

        fire_inputs(0, 0)
        drain_inputs(0)
        compute_idx(0)
        fire_inputs(1, 1)
        fire_gathers(0)

        drain_inputs(1)
        compute_idx(1)
        fire_inputs(2, 0)
        drain_gathers(0)
        fire_wb(0, 0)
        fire_gathers(1)

        def body(t, carry):
            g0 = t * _NBUF
            steady(g0, 0)
            steady(g0 + 1, 1)
            return carry

        lax.fori_loop(1, pairs, body, 0)

        drain_gathers(1)
        fire_wb(steps - 1, 1)
        drain_wb(0)
        drain_wb(1)
        drain_inputs(0)

    return lookup


def kernel(token_type_ids, sem_ids, seq_mask, emb_weight):
    b, l = token_type_ids.shape
    n = b * l
    tt = token_type_ids.reshape(n).astype(jnp.int32)
    sem = sem_ids.reshape(n).astype(jnp.int32)
    mk = seq_mask.reshape(n).astype(jnp.int32)
    out = _make_lookup(n)(tt, sem, mk, emb_weight)
    return out.reshape(b, l, EMBED_DIM)


# speedup vs baseline: 15.1380x; 8.0537x over previous
import functools

import jax
import jax.numpy as jnp
from jax import lax
from jax.experimental import pallas as pl
from jax.experimental.pallas import tpu as pltpu
from jax.experimental.pallas import tpu_sc as plsc

NUM_EMBEDDINGS = 100000
EMBED_DIM = 64
PADDING_IDX = NUM_EMBEDDINGS * 4

_LANES = 16
_NW = 32
_CHUNK = 512
_GATHER = 128
_NGATHER = _CHUNK // _GATHER
_NBUF = 2


def _make_lookup(n):
    assert n % (_NW * _CHUNK * _NBUF) == 0
    per_w = n // _NW
    steps = per_w // _CHUNK
    pairs = steps // _NBUF
    mesh = plsc.VectorSubcoreMesh(core_axis_name="c", subcore_axis_name="s")

    @functools.partial(
        pl.kernel,
        mesh=mesh,
        out_type=jax.ShapeDtypeStruct((n, EMBED_DIM), jnp.float32),
        scratch_types=[
            pltpu.VMEM((_NBUF, _CHUNK), jnp.int32),
            pltpu.VMEM((_NBUF, _CHUNK), jnp.int32),
            pltpu.VMEM((_NBUF, _CHUNK), jnp.int32),
            pltpu.VMEM((_NBUF, _NGATHER, _GATHER), jnp.int32),
            pltpu.VMEM((_NBUF, _CHUNK), jnp.float32),
            pltpu.VMEM((_NBUF, _CHUNK, EMBED_DIM), jnp.float32),
            pltpu.SemaphoreType.DMA,
            pltpu.SemaphoreType.DMA,
            pltpu.SemaphoreType.DMA,
            pltpu.SemaphoreType.DMA,
            pltpu.SemaphoreType.DMA,
            pltpu.SemaphoreType.DMA,
        ],
        compiler_params=pltpu.CompilerParams(use_tc_tiling_on_sc=False),
    )
    def lookup(tt_h, sem_h, mk_h, tab_h, out_h,
               tt_v, sem_v, mk_v, idx_v, mkf_v, rows_v,
               in_s0, in_s1, g_s0, g_s1, w_s0, w_s1):
        in_s = [in_s0, in_s1]
        g_s = [g_s0, g_s1]
        w_s = [w_s0, w_s1]
        wid = lax.axis_index("s") * 2 + lax.axis_index("c")
        wbase = wid * per_w

        def fire_inputs(g, b):
            base = wbase + g * _CHUNK
            pltpu.async_copy(tt_h.at[pl.ds(base, _CHUNK)], tt_v.at[b], in_s[b])
            pltpu.async_copy(sem_h.at[pl.ds(base, _CHUNK)], sem_v.at[b], in_s[b])
            pltpu.async_copy(mk_h.at[pl.ds(base, _CHUNK)], mk_v.at[b], in_s[b])

        def drain_inputs(b):
            sl = pl.ds(wbase, _CHUNK)
            pltpu.make_async_copy(tt_h.at[sl], tt_v.at[b], in_s[b]).wait()
            pltpu.make_async_copy(sem_h.at[sl], sem_v.at[b], in_s[b]).wait()
            pltpu.make_async_copy(mk_h.at[sl], mk_v.at[b], in_s[b]).wait()

        def compute_idx(b):
            for i in range(_CHUNK // _LANES):
                sl = pl.ds(i * _LANES, _LANES)
                ids = tt_v[b, sl] * NUM_EMBEDDINGS + sem_v[b, sl]
                idx_v[b, i // (_GATHER // _LANES),
                      pl.ds((i % (_GATHER // _LANES)) * _LANES, _LANES)] = ids
                mkf_v[b, sl] = mk_v[b, sl].astype(jnp.float32)

        def zero_masked(b):
            def zbody(k, c):
                m16 = mkf_v[b, pl.ds(k * _LANES, _LANES)]
                for j in range(_LANES):
                    r = k * _LANES + j
                    m = m16.at[jnp.full((_LANES,), j, jnp.int32)].get(
                        mode="promise_in_bounds")
                    for q in range(EMBED_DIM // _LANES):
                        sl = pl.ds(q * _LANES, _LANES)
                        rows_v[b, r, sl] = rows_v[b, r, sl] * m
                return c
            lax.fori_loop(0, _CHUNK // _LANES, zbody, 0)

        def fire_gathers(b):
            for j in range(_NGATHER):
                pltpu.async_copy(
                    tab_h.at[idx_v.at[b, j]],
                    rows_v.at[b, pl.ds(j * _GATHER, _GATHER)],
                    g_s[b],
                )

        def drain_gathers(b):
            for j in range(_NGATHER):
                pltpu.make_async_copy(
                    tab_h.at[idx_v.at[b, j]],
                    rows_v.at[b, pl.ds(j * _GATHER, _GATHER)],
                    g_s[b],
                ).wait()

        def fire_wb(g, b):
            base = wbase + g * _CHUNK
            pltpu.async_copy(rows_v.at[b], out_h.at[pl.ds(base, _CHUNK)], w_s[b])

        def drain_wb(b):
            pltpu.make_async_copy(
                rows_v.at[b], out_h.at[pl.ds(wbase, _CHUNK)], w_s[b]).wait()

        def steady(g, b):
            b1 = 1 - b
            drain_inputs(b)
            compute_idx(b)
            gnext = jnp.where(g + 1 < steps, g + 1, 0)
            fire_inputs(gnext, b1)
            drain_gathers(b1)
            zero_masked(b1)
            fire_wb(g - 1, b1)
            drain_wb(b)
            fire_gathers(b)

        fire_inputs(0, 0)
        drain_inputs(0)
        compute_idx(0)
        fire_inputs(1, 1)
        fire_gathers(0)

        drain_inputs(1)
        compute_idx(1)
        fire_inputs(2, 0)
        drain_gathers(0)
        zero_masked(0)
        fire_wb(0, 0)
        fire_gathers(1)

        def body(t, carry):
            g0 = t * _NBUF
            steady(g0, 0)
            steady(g0 + 1, 1)
            return carry

        lax.fori_loop(1, pairs, body, 0)

        drain_gathers(1)
        zero_masked(1)
        fire_wb(steps - 1, 1)
        drain_wb(0)
        drain_wb(1)
        drain_inputs(0)

    return lookup


def kernel(token_type_ids, sem_ids, seq_mask, emb_weight):
    b, l = token_type_ids.shape
    n = b * l
    tt = token_type_ids.reshape(n).astype(jnp.int32)
    sem = sem_ids.reshape(n).astype(jnp.int32)
    mk = seq_mask.reshape(n).astype(jnp.int32)
    out = _make_lookup(n)(tt, sem, mk, emb_weight)
    return out.reshape(b, l, EMBED_DIM)


# speedup vs baseline: 15.1978x; 1.0040x over previous
import functools

import jax
import jax.numpy as jnp
from jax import lax
from jax.experimental import pallas as pl
from jax.experimental.pallas import tpu as pltpu
from jax.experimental.pallas import tpu_sc as plsc

NUM_EMBEDDINGS = 100000
EMBED_DIM = 64
PADDING_IDX = NUM_EMBEDDINGS * 4

_LANES = 16
_NW = 32
_CHUNK = 640
_GATHER = 128
_NGATHER = _CHUNK // _GATHER
_NBUF = 2


def _make_lookup(n):
    assert n % (_NW * _CHUNK * _NBUF) == 0
    per_w = n // _NW
    steps = per_w // _CHUNK
    pairs = steps // _NBUF
    mesh = plsc.VectorSubcoreMesh(core_axis_name="c", subcore_axis_name="s")

    @functools.partial(
        pl.kernel,
        mesh=mesh,
        out_type=jax.ShapeDtypeStruct((n, EMBED_DIM), jnp.float32),
        scratch_types=[
            pltpu.VMEM((_NBUF, _CHUNK), jnp.int32),
            pltpu.VMEM((_NBUF, _CHUNK), jnp.int32),
            pltpu.VMEM((_NBUF, _CHUNK), jnp.int32),
            pltpu.VMEM((_NBUF, _NGATHER, _GATHER), jnp.int32),
            pltpu.VMEM((_NBUF, _CHUNK), jnp.float32),
            pltpu.VMEM((_NBUF, _CHUNK, EMBED_DIM), jnp.float32),
            pltpu.SemaphoreType.DMA,
            pltpu.SemaphoreType.DMA,
            pltpu.SemaphoreType.DMA,
            pltpu.SemaphoreType.DMA,
            pltpu.SemaphoreType.DMA,
            pltpu.SemaphoreType.DMA,
        ],
        compiler_params=pltpu.CompilerParams(use_tc_tiling_on_sc=False),
    )
    def lookup(tt_h, sem_h, mk_h, tab_h, out_h,
               tt_v, sem_v, mk_v, idx_v, mkf_v, rows_v,
               in_s0, in_s1, g_s0, g_s1, w_s0, w_s1):
        in_s = [in_s0, in_s1]
        g_s = [g_s0, g_s1]
        w_s = [w_s0, w_s1]
        wid = lax.axis_index("s") * 2 + lax.axis_index("c")
        wbase = wid * per_w

        def fire_inputs(g, b):
            base = wbase + g * _CHUNK
            pltpu.async_copy(tt_h.at[pl.ds(base, _CHUNK)], tt_v.at[b], in_s[b])
            pltpu.async_copy(sem_h.at[pl.ds(base, _CHUNK)], sem_v.at[b], in_s[b])
            pltpu.async_copy(mk_h.at[pl.ds(base, _CHUNK)], mk_v.at[b], in_s[b])

        def drain_inputs(b):
            sl = pl.ds(wbase, _CHUNK)
            pltpu.make_async_copy(tt_h.at[sl], tt_v.at[b], in_s[b]).wait()
            pltpu.make_async_copy(sem_h.at[sl], sem_v.at[b], in_s[b]).wait()
            pltpu.make_async_copy(mk_h.at[sl], mk_v.at[b], in_s[b]).wait()

        def compute_idx(b):
            for i in range(_CHUNK // _LANES):
                sl = pl.ds(i * _LANES, _LANES)
                ids = tt_v[b, sl] * NUM_EMBEDDINGS + sem_v[b, sl]
                idx_v[b, i // (_GATHER // _LANES),
                      pl.ds((i % (_GATHER // _LANES)) * _LANES, _LANES)] = ids
                mkf_v[b, sl] = mk_v[b, sl].astype(jnp.float32)

        def zero_masked(b):
            def zbody(k, c):
                m16 = mkf_v[b, pl.ds(k * _LANES, _LANES)]
                for j in range(_LANES):
                    r = k * _LANES + j
                    m = m16.at[jnp.full((_LANES,), j, jnp.int32)].get(
                        mode="promise_in_bounds")
                    for q in range(EMBED_DIM // _LANES):
                        sl = pl.ds(q * _LANES, _LANES)
                        rows_v[b, r, sl] = rows_v[b, r, sl] * m
                return c
            lax.fori_loop(0, _CHUNK // _LANES, zbody, 0)

        def fire_gathers(b):
            for j in range(_NGATHER):
                pltpu.async_copy(
                    tab_h.at[idx_v.at[b, j]],
                    rows_v.at[b, pl.ds(j * _GATHER, _GATHER)],
                    g_s[b],
                )

        def drain_gathers(b):
            for j in range(_NGATHER):
                pltpu.make_async_copy(
                    tab_h.at[idx_v.at[b, j]],
                    rows_v.at[b, pl.ds(j * _GATHER, _GATHER)],
                    g_s[b],
                ).wait()

        def fire_wb(g, b):
            base = wbase + g * _CHUNK
            pltpu.async_copy(rows_v.at[b], out_h.at[pl.ds(base, _CHUNK)], w_s[b])

        def drain_wb(b):
            pltpu.make_async_copy(
                rows_v.at[b], out_h.at[pl.ds(wbase, _CHUNK)], w_s[b]).wait()

        def steady(g, b):
            b1 = 1 - b
            drain_inputs(b)
            compute_idx(b)
            gnext = jnp.where(g + 1 < steps, g + 1, 0)
            fire_inputs(gnext, b1)
            drain_gathers(b1)
            zero_masked(b1)
            fire_wb(g - 1, b1)
            drain_wb(b)
            fire_gathers(b)

        fire_inputs(0, 0)
        drain_inputs(0)
        compute_idx(0)
        fire_inputs(1, 1)
        fire_gathers(0)

        drain_inputs(1)
        compute_idx(1)
        fire_inputs(2, 0)
        drain_gathers(0)
        zero_masked(0)
        fire_wb(0, 0)
        fire_gathers(1)

        def body(t, carry):
            g0 = t * _NBUF
            steady(g0, 0)
            steady(g0 + 1, 1)
            return carry

        lax.fori_loop(1, pairs, body, 0)

        drain_gathers(1)
        zero_masked(1)
        fire_wb(steps - 1, 1)
        drain_wb(0)
        drain_wb(1)
        drain_inputs(0)

    return lookup


def kernel(token_type_ids, sem_ids, seq_mask, emb_weight):
    b, l = token_type_ids.shape
    n = b * l
    tt = token_type_ids.reshape(n).astype(jnp.int32)
    sem = sem_ids.reshape(n).astype(jnp.int32)
    mk = seq_mask.reshape(n).astype(jnp.int32)
    out = _make_lookup(n)(tt, sem, mk, emb_weight)
    return out.reshape(b, l, EMBED_DIM)


# speedup vs baseline: 16.0871x; 1.0585x over previous
import functools

import jax
import jax.numpy as jnp
from jax import lax
from jax.experimental import pallas as pl
from jax.experimental.pallas import tpu as pltpu
from jax.experimental.pallas import tpu_sc as plsc

NUM_EMBEDDINGS = 100000
EMBED_DIM = 64
PADDING_IDX = NUM_EMBEDDINGS * 4

_LANES = 16
_NW = 32
_CHUNK = 640


def _make_lookup(n):
    assert n % (_NW * _CHUNK * 4) == 0
    per_w = n // _NW
    steps = per_w // _CHUNK
    pairs = steps // 2
    mesh = plsc.VectorSubcoreMesh(core_axis_name="c", subcore_axis_name="s")

    @functools.partial(
        pl.kernel,
        mesh=mesh,
        out_type=jax.ShapeDtypeStruct((n, EMBED_DIM), jnp.float32),
        scratch_types=[
            pltpu.VMEM((2, 3, 2 * _CHUNK), jnp.int32),
            pltpu.VMEM((2, _CHUNK), jnp.int32),
            pltpu.VMEM((2, _CHUNK), jnp.float32),
            pltpu.VMEM((2, _CHUNK, EMBED_DIM), jnp.float32),
            pltpu.SemaphoreType.DMA,
            pltpu.SemaphoreType.DMA,
            pltpu.SemaphoreType.DMA,
            pltpu.SemaphoreType.DMA,
            pltpu.SemaphoreType.DMA,
            pltpu.SemaphoreType.DMA,
        ],
        compiler_params=pltpu.CompilerParams(use_tc_tiling_on_sc=False),
    )
    def lookup(tt_h, sem_h, mk_h, tab_h, out_h,
               st_v, idx_v, mkf_v, rows_v,
               in_s0, in_s1, g_s0, g_s1, w_s0, w_s1):
        in_s = [in_s0, in_s1]
        g_s = [g_s0, g_s1]
        w_s = [w_s0, w_s1]
        wid = lax.axis_index("s") * 2 + lax.axis_index("c")
        wbase = wid * per_w

        def fire_pair_inputs(t, p):
            base = wbase + t * (2 * _CHUNK)
            sl = pl.ds(base, 2 * _CHUNK)
            pltpu.async_copy(tt_h.at[sl], st_v.at[p, 0], in_s[p])
            pltpu.async_copy(sem_h.at[sl], st_v.at[p, 1], in_s[p])
            pltpu.async_copy(mk_h.at[sl], st_v.at[p, 2], in_s[p])

        def drain_pair_inputs(p):
            sl = pl.ds(wbase, 2 * _CHUNK)
            pltpu.make_async_copy(tt_h.at[sl], st_v.at[p, 0], in_s[p]).wait()
            pltpu.make_async_copy(sem_h.at[sl], st_v.at[p, 1], in_s[p]).wait()
            pltpu.make_async_copy(mk_h.at[sl], st_v.at[p, 2], in_s[p]).wait()

        def compute_idx(p, b):
            off = b * _CHUNK
            for i in range(_CHUNK // _LANES):
                src = pl.ds(off + i * _LANES, _LANES)
                dst = pl.ds(i * _LANES, _LANES)
                idx_v[b, dst] = st_v[p, 0, src] * NUM_EMBEDDINGS + st_v[p, 1, src]
                mkf_v[b, dst] = st_v[p, 2, src].astype(jnp.float32)

        def zero_masked(b):
            def zbody(k, c):
                m16 = mkf_v[b, pl.ds(k * _LANES, _LANES)]
                for j in range(_LANES):
                    r = k * _LANES + j
                    m = m16.at[jnp.full((_LANES,), j, jnp.int32)].get(
                        mode="promise_in_bounds")
                    for q in range(EMBED_DIM // _LANES):
                        sl = pl.ds(q * _LANES, _LANES)
                        rows_v[b, r, sl] = rows_v[b, r, sl] * m
                return c
            lax.fori_loop(0, _CHUNK // _LANES, zbody, 0)

        def fire_gather(b):
            pltpu.async_copy(tab_h.at[idx_v.at[b]], rows_v.at[b], g_s[b])

        def drain_gather(b):
            pltpu.make_async_copy(
                tab_h.at[idx_v.at[b]], rows_v.at[b], g_s[b]).wait()

        def fire_wb(g, b):
            base = wbase + g * _CHUNK
            pltpu.async_copy(rows_v.at[b], out_h.at[pl.ds(base, _CHUNK)], w_s[b])

        def drain_wb(b):
            pltpu.make_async_copy(
                rows_v.at[b], out_h.at[pl.ds(wbase, _CHUNK)], w_s[b]).wait()

        def half_step(g, p, b):
            b1 = 1 - b
            compute_idx(p, b)
            drain_wb(b)
            fire_gather(b)
            drain_gather(b1)
            zero_masked(b1)
            fire_wb(g - 1, b1)

        def pair(t, p):
            drain_pair_inputs(p)
            tnext = jnp.where(t + 1 < pairs, t + 1, 0)
            fire_pair_inputs(tnext, 1 - p)
            half_step(2 * t, p, 0)
            half_step(2 * t + 1, p, 1)

        fire_pair_inputs(0, 0)
        drain_pair_inputs(0)
        fire_pair_inputs(1, 1)
        compute_idx(0, 0)
        fire_gather(0)
        compute_idx(0, 1)
        fire_gather(1)
        drain_gather(0)
        zero_masked(0)
        fire_wb(0, 0)
        pair(1, 1)

        def body(u, carry):
            pair(2 * u, 0)
            pair(2 * u + 1, 1)
            return carry

        lax.fori_loop(1, pairs // 2, body, 0)

        drain_gather(1)
        zero_masked(1)
        fire_wb(steps - 1, 1)
        drain_wb(0)
        drain_wb(1)
        drain_pair_inputs(0)

    return lookup


def kernel(token_type_ids, sem_ids, seq_mask, emb_weight):
    b, l = token_type_ids.shape
    n = b * l
    tt = token_type_ids.reshape(n).astype(jnp.int32)
    sem = sem_ids.reshape(n).astype(jnp.int32)
    mk = seq_mask.reshape(n).astype(jnp.int32)
    out = _make_lookup(n)(tt, sem, mk, emb_weight)
    return out.reshape(b, l, EMBED_DIM)


# speedup vs baseline: 18.4934x; 1.1496x over previous
import functools

import jax
import jax.numpy as jnp
from jax import lax
from jax.experimental import pallas as pl
from jax.experimental.pallas import tpu as pltpu
from jax.experimental.pallas import tpu_sc as plsc

NUM_EMBEDDINGS = 100000
EMBED_DIM = 64
PADDING_IDX = NUM_EMBEDDINGS * 4

_LANES = 16
_NW = 32
_CHUNK = 640


def _make_lookup(n):
    assert n % (_NW * _CHUNK * 4) == 0
    per_w = n // _NW
    steps = per_w // _CHUNK
    pairs = steps // 2
    mesh = plsc.VectorSubcoreMesh(core_axis_name="c", subcore_axis_name="s")

    @functools.partial(
        pl.kernel,
        mesh=mesh,
        out_type=jax.ShapeDtypeStruct((n, EMBED_DIM), jnp.float32),
        scratch_types=[
            pltpu.VMEM((2, 3, 2 * _CHUNK), jnp.int32),
            pltpu.VMEM((2, _CHUNK), jnp.int32),
            pltpu.VMEM((2, _CHUNK), jnp.float32),
            pltpu.VMEM((2, _CHUNK, EMBED_DIM), jnp.float32),
            pltpu.SemaphoreType.DMA,
            pltpu.SemaphoreType.DMA,
            pltpu.SemaphoreType.DMA,
            pltpu.SemaphoreType.DMA,
            pltpu.SemaphoreType.DMA,
            pltpu.SemaphoreType.DMA,
        ],
        compiler_params=pltpu.CompilerParams(use_tc_tiling_on_sc=False),
    )
    def lookup(pk_h, tab_h, out_h,
               st_v, idx_v, mkf_v, rows_v,
               in_s0, in_s1, g_s0, g_s1, w_s0, w_s1):
        in_s = [in_s0, in_s1]
        g_s = [g_s0, g_s1]
        w_s = [w_s0, w_s1]
        wid = lax.axis_index("s") * 2 + lax.axis_index("c")
        wbase = wid * per_w

        def fire_pair_inputs(t, p):
            base = wbase + t * (2 * _CHUNK)
            pltpu.async_copy(
                pk_h.at[:, pl.ds(base, 2 * _CHUNK)], st_v.at[p], in_s[p])

        def drain_pair_inputs(p):
            pltpu.make_async_copy(
                pk_h.at[:, pl.ds(wbase, 2 * _CHUNK)], st_v.at[p], in_s[p]).wait()

        def compute_idx(p, b):
            off = b * _CHUNK
            for i in range(_CHUNK // _LANES):
                src = pl.ds(off + i * _LANES, _LANES)
                dst = pl.ds(i * _LANES, _LANES)
                idx_v[b, dst] = st_v[p, 0, src] * NUM_EMBEDDINGS + st_v[p, 1, src]
                mkf_v[b, dst] = st_v[p, 2, src].astype(jnp.float32)

        def zero_masked(b):
            @plsc.parallel_loop(0, _CHUNK // _LANES)
            def zbody(k):
                m16 = mkf_v[b, pl.ds(k * _LANES, _LANES)]
                for j in range(_LANES):
                    r = k * _LANES + j
                    m = m16.at[jnp.full((_LANES,), j, jnp.int32)].get(
                        mode="promise_in_bounds")
                    for q in range(EMBED_DIM // _LANES):
                        sl = pl.ds(q * _LANES, _LANES)
                        rows_v[b, r, sl] = rows_v[b, r, sl] * m

        def fire_gather(b):
            pltpu.async_copy(tab_h.at[idx_v.at[b]], rows_v.at[b], g_s[b])

        def drain_gather(b):
            pltpu.make_async_copy(
                tab_h.at[idx_v.at[b]], rows_v.at[b], g_s[b]).wait()

        def fire_wb(g, b):
            base = wbase + g * _CHUNK
            pltpu.async_copy(rows_v.at[b], out_h.at[pl.ds(base, _CHUNK)], w_s[b])

        def drain_wb(b):
            pltpu.make_async_copy(
                rows_v.at[b], out_h.at[pl.ds(wbase, _CHUNK)], w_s[b]).wait()

        def half_step(g, p, b):
            b1 = 1 - b
            compute_idx(p, b)
            drain_wb(b)
            fire_gather(b)
            drain_gather(b1)
            zero_masked(b1)
            fire_wb(g - 1, b1)

        def pair(t, p):
            drain_pair_inputs(p)
            tnext = jnp.where(t + 1 < pairs, t + 1, 0)
            fire_pair_inputs(tnext, 1 - p)
            half_step(2 * t, p, 0)
            half_step(2 * t + 1, p, 1)

        fire_pair_inputs(0, 0)
        drain_pair_inputs(0)
        fire_pair_inputs(1, 1)
        compute_idx(0, 0)
        fire_gather(0)
        compute_idx(0, 1)
        fire_gather(1)
        drain_gather(0)
        zero_masked(0)
        fire_wb(0, 0)
        pair(1, 1)

        def body(u, carry):
            pair(2 * u, 0)
            pair(2 * u + 1, 1)
            return carry

        lax.fori_loop(1, pairs // 2, body, 0)

        drain_gather(1)
        zero_masked(1)
        fire_wb(steps - 1, 1)
        drain_wb(0)
        drain_wb(1)
        drain_pair_inputs(0)

    return lookup


def kernel(token_type_ids, sem_ids, seq_mask, emb_weight):
    b, l = token_type_ids.shape
    n = b * l
    packed = jnp.stack([
        token_type_ids.reshape(n).astype(jnp.int32),
        sem_ids.reshape(n).astype(jnp.int32),
        seq_mask.reshape(n).astype(jnp.int32),
    ])
    out = _make_lookup(n)(packed, emb_weight)
    return out.reshape(b, l, EMBED_DIM)
